# baseline (speedup 1.0000x reference)
import functools

import jax
import jax.numpy as jnp
from jax import lax
from jax.experimental import pallas as pl
from jax.experimental.pallas import tpu as pltpu
from jax.experimental.pallas import tpu_sc as plsc

N = 10000
D1 = 128
D2 = 64
E = 320000
NC = 2
NS = 16
CH = 128
C0 = 120
C1 = 40
NCHMAX = C0
NROW = NS * (C0 + C1)
NROWP = NS * C0 + NS * C1 + (NCHMAX - C1)
EPAD = NROW * CH
R = 10112
RPT = R // NS

_mesh = plsc.VectorSubcoreMesh(core_axis_name="c", subcore_axis_name="s")


def _zero_bf16(ref, rows, width):
    z = jnp.zeros((32,), jnp.bfloat16)
    for i in range(rows):
        for k in range(width // 32):
            ref[i, pl.ds(k * 32, 32)] = z


def _chunk_start(c, s, c0, c1):
    return pl.multiple_of(jnp.where(c == 1, s * c0, NS * c0 + s * c1), 8)


def _n_chunks(c, c0, c1):
    return jnp.where(c == 1, c0, c1)


def _deg_body(dst_hbm, deg_out, idx_v, ones_v, zb, deg_sh):
    c = lax.axis_index("c")
    s = lax.axis_index("s")

    pltpu.sync_copy(dst_hbm.at[pl.ds(_chunk_start(c, s, C0, C1), C0)], idx_v)
    one = jnp.ones((16,), jnp.float32)
    zero = jnp.zeros((16,), jnp.float32)
    for k in range(CH // 16):
        ones_v[pl.ds(k * 16, 16)] = one
    for k in range(640 // 16):
        zb[pl.ds(k * 16, 16)] = zero

    row0 = pl.multiple_of(s * RPT, 8)
    pltpu.sync_copy(zb.at[pl.ds(0, RPT)], deg_sh.at[pl.ds(row0, RPT)])
    plsc.subcore_barrier()

    def body(j, carry):
        pltpu.sync_copy(ones_v, deg_sh.at[idx_v.at[j]], add=True)
        return carry

    lax.fori_loop(0, _n_chunks(c, C0, C1), body, 0)
    plsc.subcore_barrier()

    pltpu.sync_copy(deg_sh.at[pl.ds(row0, RPT)], zb.at[pl.ds(0, RPT)])
    pltpu.sync_copy(zb.at[pl.ds(0, RPT)], deg_out.at[c, pl.ds(row0, RPT)])


_deg_call = functools.partial(
    pl.kernel,
    out_type=jax.ShapeDtypeStruct((NC, R), jnp.float32),
    mesh=_mesh,
    scratch_types=[
        pltpu.VMEM((C0, CH), jnp.int32),
        pltpu.VMEM((CH,), jnp.float32),
        pltpu.VMEM((640,), jnp.float32),
        pltpu.VMEM_SHARED((R,), jnp.float32),
    ],
    compiler_params=pltpu.CompilerParams(use_tc_tiling_on_sc=False),
)(_deg_body)


def _agg_body(D, c0, c1, nout, g_hbm, src_hbm, dst_hbm, acc_out, sidx, didx,
              rows, zb, acc_sh, semg, semg2):
    c = lax.axis_index("c")
    s = lax.axis_index("s")
    row0 = pl.multiple_of(s * RPT, 8)

    def work():
        start = _chunk_start(c, s, c0, c1)
        pltpu.sync_copy(src_hbm.at[pl.ds(start, c0)], sidx)
        pltpu.sync_copy(dst_hbm.at[pl.ds(start, c0)], didx)

        _zero_bf16(zb, 16, D)
        def zbody(k, carry):
            pltpu.sync_copy(zb, acc_sh.at[pl.ds(row0 + k * 16, 16)])
            return carry
        lax.fori_loop(0, RPT // 16, zbody, 0)
        pltpu.sync_copy(zb.at[pl.ds(0, RPT % 16)],
                        acc_sh.at[pl.ds(row0 + (RPT // 16) * 16, RPT % 16)])

    if c1 > 0:
        work()
    else:
        pl.when(c == 1)(work)
    plsc.subcore_barrier()

    def body(k, carry):
        j0 = 2 * k
        j1 = j0 + 1
        cp0 = pltpu.async_copy(g_hbm.at[sidx.at[j0]], rows.at[0], semg)
        cp1 = pltpu.async_copy(g_hbm.at[sidx.at[j1]], rows.at[1], semg2)
        cp0.wait()
        pltpu.sync_copy(rows.at[0], acc_sh.at[didx.at[j0]], add=True)
        cp1.wait()
        pltpu.sync_copy(rows.at[1], acc_sh.at[didx.at[j1]], add=True)
        return carry

    lax.fori_loop(0, _n_chunks(c, c0, c1) // 2, body, 0)
    plsc.subcore_barrier()

    def copyout():
        def obody(k, carry):
            pltpu.sync_copy(acc_sh.at[pl.ds(row0 + k * CH, CH)], rows.at[0])
            if nout == 1:
                pltpu.sync_copy(rows.at[0],
                                acc_out.at[pl.ds(row0 + k * CH, CH)])
            else:
                pltpu.sync_copy(rows.at[0],
                                acc_out.at[c, pl.ds(row0 + k * CH, CH)])
            return carry
        lax.fori_loop(0, RPT // CH, obody, 0)
        tail = RPT % CH
        trow = row0 + (RPT // CH) * CH
        pltpu.sync_copy(acc_sh.at[pl.ds(trow, tail)], rows.at[0, pl.ds(0, tail)])
        if nout == 1:
            pltpu.sync_copy(rows.at[0, pl.ds(0, tail)],
                            acc_out.at[pl.ds(trow, tail)])
        else:
            pltpu.sync_copy(rows.at[0, pl.ds(0, tail)],
                            acc_out.at[c, pl.ds(trow, tail)])

    if c1 > 0:
        copyout()
    else:
        pl.when(c == 1)(copyout)


def _make_agg(D, c0, c1, nout):
    oshape = (R, D) if nout == 1 else (NC, R, D)
    return functools.partial(
        pl.kernel,
        out_type=jax.ShapeDtypeStruct(oshape, jnp.bfloat16),
        mesh=_mesh,
        scratch_types=[
            pltpu.VMEM((c0, CH), jnp.int32),
            pltpu.VMEM((c0, CH), jnp.int32),
            pltpu.VMEM((2, CH, D), jnp.bfloat16),
            pltpu.VMEM((16, D), jnp.bfloat16),
            pltpu.VMEM_SHARED((R, D), jnp.bfloat16),
            pltpu.SemaphoreType.DMA,
            pltpu.SemaphoreType.DMA,
        ],
        compiler_params=pltpu.CompilerParams(use_tc_tiling_on_sc=False),
    )(functools.partial(_agg_body, D, c0, c1, nout))


_agg128 = _make_agg(D1, 136, 24, NC)
_agg64 = _make_agg(D2, C0, C1, NC)


def _t0_body(deg_ref, dis_ref):
    deg = deg_ref[0:1, :] + deg_ref[1:2, :] + 1.0
    dis_ref[...] = lax.rsqrt(deg)


def _t0(deg2):
    return pl.pallas_call(
        _t0_body,
        out_shape=jax.ShapeDtypeStruct((1, R), jnp.float32),
    )(deg2)


_BR = 1000
_NB = N // _BR


def _t1_body(x_ref, w_ref, dis_ref, o_ref):
    mm = jnp.dot(x_ref[...], w_ref[...],
                 preferred_element_type=jnp.float32) * dis_ref[...]
    o_ref[...] = mm.astype(jnp.bfloat16)


def _t1(x, W1, dis_col):
    return pl.pallas_call(
        _t1_body,
        grid=(_NB,),
        in_specs=[
            pl.BlockSpec((_BR, D1), lambda i: (i, 0)),
            pl.BlockSpec((D1, D1), lambda i: (0, 0)),
            pl.BlockSpec((_BR, 1), lambda i: (i, 0)),
        ],
        out_specs=pl.BlockSpec((_BR, D1), lambda i: (i, 0)),
        out_shape=jax.ShapeDtypeStruct((N, D1), jnp.bfloat16),
    )(x, W1, dis_col)


def _t2_body(p_ref, g_ref, dis_ref, b_ref, w_ref, o_ref):
    dis = dis_ref[...]
    acc = p_ref[0].astype(jnp.float32) + p_ref[1].astype(jnp.float32)
    z = dis * (acc + g_ref[...].astype(jnp.float32)) + b_ref[...]
    h = jnp.maximum(z, 0.0)
    mm = jnp.dot(h, w_ref[...], preferred_element_type=jnp.float32) * dis
    o_ref[...] = mm.astype(jnp.bfloat16)


def _t2(p1, g1, dis_col, b1, W2):
    return pl.pallas_call(
        _t2_body,
        grid=(_NB,),
        in_specs=[
            pl.BlockSpec((NC, _BR, D1), lambda i: (0, i, 0)),
            pl.BlockSpec((_BR, D1), lambda i: (i, 0)),
            pl.BlockSpec((_BR, 1), lambda i: (i, 0)),
            pl.BlockSpec((1, D1), lambda i: (0, 0)),
            pl.BlockSpec((D1, D2), lambda i: (0, 0)),
        ],
        out_specs=pl.BlockSpec((_BR, D2), lambda i: (i, 0)),
        out_shape=jax.ShapeDtypeStruct((N, D2), jnp.bfloat16),
    )(p1, g1, dis_col, b1.reshape(1, D1), W2)


def _t3_body(p_ref, g_ref, dis_ref, b_ref, o_ref):
    acc = (p_ref[0] + p_ref[1]).astype(jnp.float32)
    z = dis_ref[...] * (acc + g_ref[...].astype(jnp.float32)) + b_ref[...]
    m = jnp.max(z, axis=-1, keepdims=True)
    zs = z - m
    o_ref[...] = zs - jnp.log(jnp.sum(jnp.exp(zs), axis=-1, keepdims=True))


def _t3(p2, g2, dis_col, b2):
    return pl.pallas_call(
        _t3_body,
        grid=(_NB,),
        in_specs=[
            pl.BlockSpec((NC, _BR, D2), lambda i: (0, i, 0)),
            pl.BlockSpec((_BR, D2), lambda i: (i, 0)),
            pl.BlockSpec((_BR, 1), lambda i: (i, 0)),
            pl.BlockSpec((1, D2), lambda i: (0, 0)),
        ],
        out_specs=pl.BlockSpec((_BR, D2), lambda i: (i, 0)),
        out_shape=jax.ShapeDtypeStruct((N, D2), jnp.float32),
    )(p2, g2, dis_col, b2.reshape(1, D2))


def _pack(flat, fill, c0, c1):
    tp = NS * c0 + NS * c1 + (c0 - c1) - NROW
    a = jnp.concatenate([flat, jnp.full((tp * CH,), fill, jnp.int32)])
    return a.reshape(NROW + tp, CH)


def kernel(x, edge_index, W1, b1, W2, b2):
    src = edge_index[0].astype(jnp.int32)
    dst = edge_index[1].astype(jnp.int32)
    pad = EPAD - E
    flat_src = jnp.concatenate([src, jnp.zeros((pad,), jnp.int32)])
    flat_dst = jnp.concatenate([dst, jnp.full((pad,), N, jnp.int32)])
    s128 = _pack(flat_src, 0, 136, 24)
    d128 = _pack(flat_dst, N, 136, 24)
    s64 = _pack(flat_src, 0, C0, C1)
    d64 = _pack(flat_dst, N, C0, C1)

    deg2 = _deg_call(d64)
    dis_col = _t0(deg2).reshape(R, 1)
    g1 = _t1(x, W1, dis_col[:N])
    p1 = _agg128(g1, s128, d128)
    g2 = _t2(p1, g1, dis_col[:N], b1, W2)
    p2 = _agg64(g2, s64, d64)
    return _t3(p2, g2, dis_col[:N], b2)


# speedup vs baseline: 1.0565x; 1.0565x over previous
import functools

import jax
import jax.numpy as jnp
from jax import lax
from jax.experimental import pallas as pl
from jax.experimental.pallas import tpu as pltpu
from jax.experimental.pallas import tpu_sc as plsc

N = 10000
D1 = 128
D2 = 64
E = 320000
NC = 2
NS = 16
CH = 128
C0 = 120
C1 = 40
NCHMAX = C0
NROW = NS * (C0 + C1)
NROWP = NS * C0 + NS * C1 + (NCHMAX - C1)
EPAD = NROW * CH
R = 10112
RPT = R // NS

_mesh = plsc.VectorSubcoreMesh(core_axis_name="c", subcore_axis_name="s")


def _zero_bf16(ref, rows, width):
    z = jnp.zeros((32,), jnp.bfloat16)
    for i in range(rows):
        for k in range(width // 32):
            ref[i, pl.ds(k * 32, 32)] = z


def _chunk_start(c, s, c0, c1):
    return pl.multiple_of(jnp.where(c == 1, s * c0, NS * c0 + s * c1), 8)


def _n_chunks(c, c0, c1):
    return jnp.where(c == 1, c0, c1)


def _deg_body(dst_hbm, deg_out, idx_v, ones_v, zb, deg_sh):
    c = lax.axis_index("c")
    s = lax.axis_index("s")

    pltpu.sync_copy(dst_hbm.at[pl.ds(_chunk_start(c, s, C0, C1), C0)], idx_v)
    one = jnp.ones((16,), jnp.float32)
    zero = jnp.zeros((16,), jnp.float32)
    for k in range(CH // 16):
        ones_v[pl.ds(k * 16, 16)] = one
    for k in range(640 // 16):
        zb[pl.ds(k * 16, 16)] = zero

    row0 = pl.multiple_of(s * RPT, 8)
    pltpu.sync_copy(zb.at[pl.ds(0, RPT)], deg_sh.at[pl.ds(row0, RPT)])
    plsc.subcore_barrier()

    def body(j, carry):
        pltpu.sync_copy(ones_v, deg_sh.at[idx_v.at[j]], add=True)
        return carry

    lax.fori_loop(0, _n_chunks(c, C0, C1), body, 0)
    plsc.subcore_barrier()

    pltpu.sync_copy(deg_sh.at[pl.ds(row0, RPT)], zb.at[pl.ds(0, RPT)])
    pltpu.sync_copy(zb.at[pl.ds(0, RPT)], deg_out.at[c, pl.ds(row0, RPT)])


_deg_call = functools.partial(
    pl.kernel,
    out_type=jax.ShapeDtypeStruct((NC, R), jnp.float32),
    mesh=_mesh,
    scratch_types=[
        pltpu.VMEM((C0, CH), jnp.int32),
        pltpu.VMEM((CH,), jnp.float32),
        pltpu.VMEM((640,), jnp.float32),
        pltpu.VMEM_SHARED((R,), jnp.float32),
    ],
    compiler_params=pltpu.CompilerParams(use_tc_tiling_on_sc=False),
)(_deg_body)


def _agg_body(D, c0, c1, nout, g_hbm, src_hbm, dst_hbm, acc_out, sidx, didx,
              rows, zb, acc_sh, semg, semg2):
    c = lax.axis_index("c")
    s = lax.axis_index("s")
    row0 = pl.multiple_of(s * RPT, 8)

    def work():
        start = _chunk_start(c, s, c0, c1)
        pltpu.sync_copy(src_hbm.at[pl.ds(start, c0)], sidx)
        pltpu.sync_copy(dst_hbm.at[pl.ds(start, c0)], didx)

        _zero_bf16(zb, 16, D)
        def zbody(k, carry):
            pltpu.sync_copy(zb, acc_sh.at[pl.ds(row0 + k * 16, 16)])
            return carry
        lax.fori_loop(0, RPT // 16, zbody, 0)
        pltpu.sync_copy(zb.at[pl.ds(0, RPT % 16)],
                        acc_sh.at[pl.ds(row0 + (RPT // 16) * 16, RPT % 16)])

    if c1 > 0:
        work()
    else:
        pl.when(c == 1)(work)
    plsc.subcore_barrier()

    def body(k, carry):
        j0 = 2 * k
        j1 = j0 + 1
        cp0 = pltpu.async_copy(g_hbm.at[sidx.at[j0]], rows.at[0], semg)
        cp1 = pltpu.async_copy(g_hbm.at[sidx.at[j1]], rows.at[1], semg2)
        cp0.wait()
        pltpu.sync_copy(rows.at[0], acc_sh.at[didx.at[j0]], add=True)
        cp1.wait()
        pltpu.sync_copy(rows.at[1], acc_sh.at[didx.at[j1]], add=True)
        return carry

    lax.fori_loop(0, _n_chunks(c, c0, c1) // 2, body, 0)
    plsc.subcore_barrier()

    def copyout():
        def obody(k, carry):
            pltpu.sync_copy(acc_sh.at[pl.ds(row0 + k * CH, CH)], rows.at[0])
            if nout == 1:
                pltpu.sync_copy(rows.at[0],
                                acc_out.at[pl.ds(row0 + k * CH, CH)])
            else:
                pltpu.sync_copy(rows.at[0],
                                acc_out.at[c, pl.ds(row0 + k * CH, CH)])
            return carry
        lax.fori_loop(0, RPT // CH, obody, 0)
        tail = RPT % CH
        trow = row0 + (RPT // CH) * CH
        pltpu.sync_copy(acc_sh.at[pl.ds(trow, tail)], rows.at[0, pl.ds(0, tail)])
        if nout == 1:
            pltpu.sync_copy(rows.at[0, pl.ds(0, tail)],
                            acc_out.at[pl.ds(trow, tail)])
        else:
            pltpu.sync_copy(rows.at[0, pl.ds(0, tail)],
                            acc_out.at[c, pl.ds(trow, tail)])

    if c1 > 0:
        copyout()
    else:
        pl.when(c == 1)(copyout)


def _make_agg(D, c0, c1, nout):
    oshape = (R, D) if nout == 1 else (NC, R, D)
    return functools.partial(
        pl.kernel,
        out_type=jax.ShapeDtypeStruct(oshape, jnp.bfloat16),
        mesh=_mesh,
        scratch_types=[
            pltpu.VMEM((c0, CH), jnp.int32),
            pltpu.VMEM((c0, CH), jnp.int32),
            pltpu.VMEM((2, CH, D), jnp.bfloat16),
            pltpu.VMEM((16, D), jnp.bfloat16),
            pltpu.VMEM_SHARED((R, D), jnp.bfloat16),
            pltpu.SemaphoreType.DMA,
            pltpu.SemaphoreType.DMA,
        ],
        compiler_params=pltpu.CompilerParams(use_tc_tiling_on_sc=False),
    )(functools.partial(_agg_body, D, c0, c1, nout))


_agg128 = _make_agg(D1, 144, 16, NC)
_agg64 = _make_agg(D2, C0, C1, NC)


def _t0_body(deg_ref, dis_ref):
    deg = deg_ref[0:1, :] + deg_ref[1:2, :] + 1.0
    dis_ref[...] = lax.rsqrt(deg)


def _t0(deg2):
    return pl.pallas_call(
        _t0_body,
        out_shape=jax.ShapeDtypeStruct((1, R), jnp.float32),
    )(deg2)


_BR = 1000
_NB = N // _BR


def _t1_body(x_ref, w_ref, dis_ref, o_ref):
    mm = jnp.dot(x_ref[...], w_ref[...],
                 preferred_element_type=jnp.float32) * dis_ref[...]
    o_ref[...] = mm.astype(jnp.bfloat16)


def _t1(x, W1, dis_col):
    return pl.pallas_call(
        _t1_body,
        grid=(_NB,),
        in_specs=[
            pl.BlockSpec((_BR, D1), lambda i: (i, 0)),
            pl.BlockSpec((D1, D1), lambda i: (0, 0)),
            pl.BlockSpec((_BR, 1), lambda i: (i, 0)),
        ],
        out_specs=pl.BlockSpec((_BR, D1), lambda i: (i, 0)),
        out_shape=jax.ShapeDtypeStruct((N, D1), jnp.bfloat16),
    )(x, W1, dis_col)


def _t2_body(p_ref, g_ref, dis_ref, b_ref, w_ref, o_ref):
    dis = dis_ref[...]
    acc = p_ref[0].astype(jnp.float32) + p_ref[1].astype(jnp.float32)
    z = dis * (acc + g_ref[...].astype(jnp.float32)) + b_ref[...]
    h = jnp.maximum(z, 0.0)
    mm = jnp.dot(h, w_ref[...], preferred_element_type=jnp.float32) * dis
    o_ref[...] = mm.astype(jnp.bfloat16)


def _t2(p1, g1, dis_col, b1, W2):
    return pl.pallas_call(
        _t2_body,
        grid=(_NB,),
        in_specs=[
            pl.BlockSpec((NC, _BR, D1), lambda i: (0, i, 0)),
            pl.BlockSpec((_BR, D1), lambda i: (i, 0)),
            pl.BlockSpec((_BR, 1), lambda i: (i, 0)),
            pl.BlockSpec((1, D1), lambda i: (0, 0)),
            pl.BlockSpec((D1, D2), lambda i: (0, 0)),
        ],
        out_specs=pl.BlockSpec((_BR, D2), lambda i: (i, 0)),
        out_shape=jax.ShapeDtypeStruct((N, D2), jnp.bfloat16),
    )(p1, g1, dis_col, b1.reshape(1, D1), W2)


def _t3_body(p_ref, g_ref, dis_ref, b_ref, o_ref):
    acc = (p_ref[0] + p_ref[1]).astype(jnp.float32)
    z = dis_ref[...] * (acc + g_ref[...].astype(jnp.float32)) + b_ref[...]
    m = jnp.max(z, axis=-1, keepdims=True)
    zs = z - m
    o_ref[...] = zs - jnp.log(jnp.sum(jnp.exp(zs), axis=-1, keepdims=True))


def _t3(p2, g2, dis_col, b2):
    return pl.pallas_call(
        _t3_body,
        grid=(_NB,),
        in_specs=[
            pl.BlockSpec((NC, _BR, D2), lambda i: (0, i, 0)),
            pl.BlockSpec((_BR, D2), lambda i: (i, 0)),
            pl.BlockSpec((_BR, 1), lambda i: (i, 0)),
            pl.BlockSpec((1, D2), lambda i: (0, 0)),
        ],
        out_specs=pl.BlockSpec((_BR, D2), lambda i: (i, 0)),
        out_shape=jax.ShapeDtypeStruct((N, D2), jnp.float32),
    )(p2, g2, dis_col, b2.reshape(1, D2))


def _pack(flat, fill, c0, c1):
    tp = NS * c0 + NS * c1 + (c0 - c1) - NROW
    a = jnp.concatenate([flat, jnp.full((tp * CH,), fill, jnp.int32)])
    return a.reshape(NROW + tp, CH)


def kernel(x, edge_index, W1, b1, W2, b2):
    src = edge_index[0].astype(jnp.int32)
    dst = edge_index[1].astype(jnp.int32)
    pad = EPAD - E
    flat_src = jnp.concatenate([src, jnp.zeros((pad,), jnp.int32)])
    flat_dst = jnp.concatenate([dst, jnp.full((pad,), N, jnp.int32)])
    s128 = _pack(flat_src, 0, 144, 16)
    d128 = _pack(flat_dst, N, 144, 16)
    s64 = _pack(flat_src, 0, C0, C1)
    d64 = _pack(flat_dst, N, C0, C1)

    deg2 = _deg_call(d64)
    dis_col = _t0(deg2).reshape(R, 1)
    g1 = _t1(x, W1, dis_col[:N])
    p1 = _agg128(g1, s128, d128)
    g2 = _t2(p1, g1, dis_col[:N], b1, W2)
    p2 = _agg64(g2, s64, d64)
    return _t3(p2, g2, dis_col[:N], b2)


# speedup vs baseline: 1.0774x; 1.0198x over previous
import functools

import jax
import jax.numpy as jnp
from jax import lax
from jax.experimental import pallas as pl
from jax.experimental.pallas import tpu as pltpu
from jax.experimental.pallas import tpu_sc as plsc

N = 10000
D1 = 128
D2 = 64
E = 320000
NC = 2
NS = 16
CH = 128
C0 = 120
C1 = 40
NCHMAX = C0
NROW = NS * (C0 + C1)
NROWP = NS * C0 + NS * C1 + (NCHMAX - C1)
EPAD = NROW * CH
R = 10112
RPT = R // NS

_mesh = plsc.VectorSubcoreMesh(core_axis_name="c", subcore_axis_name="s")


def _zero_bf16(ref, rows, width):
    z = jnp.zeros((32,), jnp.bfloat16)
    for i in range(rows):
        for k in range(width // 32):
            ref[i, pl.ds(k * 32, 32)] = z


def _chunk_start(c, s, c0, c1):
    return pl.multiple_of(jnp.where(c == 1, s * c0, NS * c0 + s * c1), 8)


def _n_chunks(c, c0, c1):
    return jnp.where(c == 1, c0, c1)


def _deg_body(dst_hbm, deg_out, idx_v, ones_v, zb, deg_sh):
    c = lax.axis_index("c")
    s = lax.axis_index("s")

    pltpu.sync_copy(dst_hbm.at[pl.ds(_chunk_start(c, s, C0, C1), C0)], idx_v)
    one = jnp.ones((16,), jnp.float32)
    zero = jnp.zeros((16,), jnp.float32)
    for k in range(CH // 16):
        ones_v[pl.ds(k * 16, 16)] = one
    for k in range(640 // 16):
        zb[pl.ds(k * 16, 16)] = zero

    row0 = pl.multiple_of(s * RPT, 8)
    pltpu.sync_copy(zb.at[pl.ds(0, RPT)], deg_sh.at[pl.ds(row0, RPT)])
    plsc.subcore_barrier()

    def body(j, carry):
        pltpu.sync_copy(ones_v, deg_sh.at[idx_v.at[j]], add=True)
        return carry

    lax.fori_loop(0, _n_chunks(c, C0, C1), body, 0)
    plsc.subcore_barrier()

    pltpu.sync_copy(deg_sh.at[pl.ds(row0, RPT)], zb.at[pl.ds(0, RPT)])
    pltpu.sync_copy(zb.at[pl.ds(0, RPT)], deg_out.at[c, pl.ds(row0, RPT)])


_deg_call = functools.partial(
    pl.kernel,
    out_type=jax.ShapeDtypeStruct((NC, R), jnp.float32),
    mesh=_mesh,
    scratch_types=[
        pltpu.VMEM((C0, CH), jnp.int32),
        pltpu.VMEM((CH,), jnp.float32),
        pltpu.VMEM((640,), jnp.float32),
        pltpu.VMEM_SHARED((R,), jnp.float32),
    ],
    compiler_params=pltpu.CompilerParams(use_tc_tiling_on_sc=False),
)(_deg_body)


def _agg_body(D, c0, c1, nout, g_hbm, src_hbm, dst_hbm, acc_out, sidx, didx,
              rows, zb, acc_sh, semg, semg2):
    c = lax.axis_index("c")
    s = lax.axis_index("s")
    row0 = pl.multiple_of(s * RPT, 8)

    def work():
        start = _chunk_start(c, s, c0, c1)
        pltpu.sync_copy(src_hbm.at[pl.ds(start, c0)], sidx)
        pltpu.sync_copy(dst_hbm.at[pl.ds(start, c0)], didx)

        _zero_bf16(zb, 16, D)
        def zbody(k, carry):
            pltpu.sync_copy(zb, acc_sh.at[pl.ds(row0 + k * 16, 16)])
            return carry
        lax.fori_loop(0, RPT // 16, zbody, 0)
        pltpu.sync_copy(zb.at[pl.ds(0, RPT % 16)],
                        acc_sh.at[pl.ds(row0 + (RPT // 16) * 16, RPT % 16)])

    if c1 > 0:
        work()
    else:
        pl.when(c == 1)(work)
    plsc.subcore_barrier()

    def body(k, carry):
        j0 = 2 * k
        j1 = j0 + 1
        cp0 = pltpu.async_copy(g_hbm.at[sidx.at[j0]], rows.at[0], semg)
        cp1 = pltpu.async_copy(g_hbm.at[sidx.at[j1]], rows.at[1], semg2)
        cp0.wait()
        pltpu.sync_copy(rows.at[0], acc_sh.at[didx.at[j0]], add=True)
        cp1.wait()
        pltpu.sync_copy(rows.at[1], acc_sh.at[didx.at[j1]], add=True)
        return carry

    lax.fori_loop(0, _n_chunks(c, c0, c1) // 2, body, 0)
    plsc.subcore_barrier()

    def copyout():
        def obody(k, carry):
            pltpu.sync_copy(acc_sh.at[pl.ds(row0 + k * CH, CH)], rows.at[0])
            if nout == 1:
                pltpu.sync_copy(rows.at[0],
                                acc_out.at[pl.ds(row0 + k * CH, CH)])
            else:
                pltpu.sync_copy(rows.at[0],
                                acc_out.at[c, pl.ds(row0 + k * CH, CH)])
            return carry
        lax.fori_loop(0, RPT // CH, obody, 0)
        tail = RPT % CH
        trow = row0 + (RPT // CH) * CH
        pltpu.sync_copy(acc_sh.at[pl.ds(trow, tail)], rows.at[0, pl.ds(0, tail)])
        if nout == 1:
            pltpu.sync_copy(rows.at[0, pl.ds(0, tail)],
                            acc_out.at[pl.ds(trow, tail)])
        else:
            pltpu.sync_copy(rows.at[0, pl.ds(0, tail)],
                            acc_out.at[c, pl.ds(trow, tail)])

    if c1 > 0:
        copyout()
    else:
        pl.when(c == 1)(copyout)


def _make_agg(D, c0, c1, nout):
    oshape = (R, D) if nout == 1 else (NC, R, D)
    return functools.partial(
        pl.kernel,
        out_type=jax.ShapeDtypeStruct(oshape, jnp.bfloat16),
        mesh=_mesh,
        scratch_types=[
            pltpu.VMEM((c0, CH), jnp.int32),
            pltpu.VMEM((c0, CH), jnp.int32),
            pltpu.VMEM((2, CH, D), jnp.bfloat16),
            pltpu.VMEM((16, D), jnp.bfloat16),
            pltpu.VMEM_SHARED((R, D), jnp.bfloat16),
            pltpu.SemaphoreType.DMA,
            pltpu.SemaphoreType.DMA,
        ],
        compiler_params=pltpu.CompilerParams(use_tc_tiling_on_sc=False),
    )(functools.partial(_agg_body, D, c0, c1, nout))


_agg128 = _make_agg(D1, 152, 8, NC)
_agg64 = _make_agg(D2, C0, C1, NC)


def _t0_body(deg_ref, dis_ref):
    deg = deg_ref[0:1, :] + deg_ref[1:2, :] + 1.0
    dis_ref[...] = lax.rsqrt(deg)


def _t0(deg2):
    return pl.pallas_call(
        _t0_body,
        out_shape=jax.ShapeDtypeStruct((1, R), jnp.float32),
    )(deg2)


_BR = 1000
_NB = N // _BR


def _t1_body(x_ref, w_ref, dis_ref, o_ref):
    mm = jnp.dot(x_ref[...], w_ref[...],
                 preferred_element_type=jnp.float32) * dis_ref[...]
    o_ref[...] = mm.astype(jnp.bfloat16)


def _t1(x, W1, dis_col):
    return pl.pallas_call(
        _t1_body,
        grid=(_NB,),
        in_specs=[
            pl.BlockSpec((_BR, D1), lambda i: (i, 0)),
            pl.BlockSpec((D1, D1), lambda i: (0, 0)),
            pl.BlockSpec((_BR, 1), lambda i: (i, 0)),
        ],
        out_specs=pl.BlockSpec((_BR, D1), lambda i: (i, 0)),
        out_shape=jax.ShapeDtypeStruct((N, D1), jnp.bfloat16),
    )(x, W1, dis_col)


def _t2_body(p_ref, g_ref, dis_ref, b_ref, w_ref, o_ref):
    dis = dis_ref[...]
    acc = p_ref[0].astype(jnp.float32) + p_ref[1].astype(jnp.float32)
    z = dis * (acc + g_ref[...].astype(jnp.float32)) + b_ref[...]
    h = jnp.maximum(z, 0.0)
    mm = jnp.dot(h, w_ref[...], preferred_element_type=jnp.float32) * dis
    o_ref[...] = mm.astype(jnp.bfloat16)


def _t2(p1, g1, dis_col, b1, W2):
    return pl.pallas_call(
        _t2_body,
        grid=(_NB,),
        in_specs=[
            pl.BlockSpec((NC, _BR, D1), lambda i: (0, i, 0)),
            pl.BlockSpec((_BR, D1), lambda i: (i, 0)),
            pl.BlockSpec((_BR, 1), lambda i: (i, 0)),
            pl.BlockSpec((1, D1), lambda i: (0, 0)),
            pl.BlockSpec((D1, D2), lambda i: (0, 0)),
        ],
        out_specs=pl.BlockSpec((_BR, D2), lambda i: (i, 0)),
        out_shape=jax.ShapeDtypeStruct((N, D2), jnp.bfloat16),
    )(p1, g1, dis_col, b1.reshape(1, D1), W2)


def _t3_body(p_ref, g_ref, dis_ref, b_ref, o_ref):
    acc = (p_ref[0] + p_ref[1]).astype(jnp.float32)
    z = dis_ref[...] * (acc + g_ref[...].astype(jnp.float32)) + b_ref[...]
    m = jnp.max(z, axis=-1, keepdims=True)
    zs = z - m
    o_ref[...] = zs - jnp.log(jnp.sum(jnp.exp(zs), axis=-1, keepdims=True))


def _t3(p2, g2, dis_col, b2):
    return pl.pallas_call(
        _t3_body,
        grid=(_NB,),
        in_specs=[
            pl.BlockSpec((NC, _BR, D2), lambda i: (0, i, 0)),
            pl.BlockSpec((_BR, D2), lambda i: (i, 0)),
            pl.BlockSpec((_BR, 1), lambda i: (i, 0)),
            pl.BlockSpec((1, D2), lambda i: (0, 0)),
        ],
        out_specs=pl.BlockSpec((_BR, D2), lambda i: (i, 0)),
        out_shape=jax.ShapeDtypeStruct((N, D2), jnp.float32),
    )(p2, g2, dis_col, b2.reshape(1, D2))


def _pack(flat, fill, c0, c1):
    tp = NS * c0 + NS * c1 + (c0 - c1) - NROW
    a = jnp.concatenate([flat, jnp.full((tp * CH,), fill, jnp.int32)])
    return a.reshape(NROW + tp, CH)


def kernel(x, edge_index, W1, b1, W2, b2):
    src = edge_index[0].astype(jnp.int32)
    dst = edge_index[1].astype(jnp.int32)
    pad = EPAD - E
    flat_src = jnp.concatenate([src, jnp.zeros((pad,), jnp.int32)])
    flat_dst = jnp.concatenate([dst, jnp.full((pad,), N, jnp.int32)])
    s128 = _pack(flat_src, 0, 152, 8)
    d128 = _pack(flat_dst, N, 152, 8)
    s64 = _pack(flat_src, 0, C0, C1)
    d64 = _pack(flat_dst, N, C0, C1)

    deg2 = _deg_call(d64)
    dis_col = _t0(deg2).reshape(R, 1)
    g1 = _t1(x, W1, dis_col[:N])
    p1 = _agg128(g1, s128, d128)
    g2 = _t2(p1, g1, dis_col[:N], b1, W2)
    p2 = _agg64(g2, s64, d64)
    return _t3(p2, g2, dis_col[:N], b2)
